# baseline (speedup 1.0000x reference)
import functools

import jax
import jax.numpy as jnp
from jax import lax
from jax.experimental import pallas as pl
from jax.experimental.pallas import tpu as pltpu
from jax.experimental.pallas import tpu_sc as plsc

_NF = 4
_FD = 16
_HID = 32
_EPS = 1e-12
_CH = 128


def _sc_gather(u2_flat, i2_flat, u1_tab, i1_tab, uidx, iidx, B, V):
    info = plsc.get_sparse_core_info()
    nw = info.num_cores * info.num_subcores
    b_per_w = B // nw
    nel = b_per_w * _HID
    mesh = plsc.VectorSubcoreMesh(core_axis_name="c", subcore_axis_name="s")

    @functools.partial(
        pl.kernel,
        mesh=mesh,
        compiler_params=pltpu.CompilerParams(use_tc_tiling_on_sc=False,
                                             needs_layout_passes=False),
        out_type=[
            jax.ShapeDtypeStruct((B * _HID,), jnp.float32),
            jax.ShapeDtypeStruct((B * _HID,), jnp.float32),
            jax.ShapeDtypeStruct((B,), jnp.float32),
            jax.ShapeDtypeStruct((B,), jnp.float32),
        ],
        scratch_types=[
            pltpu.VMEM((b_per_w,), jnp.int32),
            pltpu.VMEM((b_per_w,), jnp.int32),
            pltpu.VMEM((nel,), jnp.int32),
            pltpu.VMEM((nel,), jnp.int32),
            pltpu.VMEM((nel,), jnp.float32),
            pltpu.VMEM((nel,), jnp.float32),
            pltpu.VMEM((b_per_w,), jnp.float32),
            pltpu.VMEM((b_per_w,), jnp.float32),
            pltpu.SemaphoreType.DMA,
            pltpu.SemaphoreType.DMA,
            pltpu.SemaphoreType.DMA,
        ],
    )
    def gath(u2_t, i2_t, u1_t, i1_t, uix_h, iix_h,
             u2_o, i2_o, u1_o, i1_o,
             uix_v, iix_v, uoff, ioff, ubuf, ibuf,
             u1_v, i1_v, usem, isem, fsem):
        wid = lax.axis_index("s") * info.num_cores + lax.axis_index("c")
        base = wid * b_per_w
        pltpu.sync_copy(uix_h.at[pl.ds(base, b_per_w)], uix_v)
        pltpu.sync_copy(iix_h.at[pl.ds(base, b_per_w)], iix_v)
        fcopies = []
        for j in range(b_per_w // _CH):
            lo = j * _CH
            fcopies.append(pltpu.async_copy(
                u1_t.at[uix_v.at[pl.ds(lo, _CH)]],
                u1_v.at[pl.ds(lo, _CH)], fsem))
            fcopies.append(pltpu.async_copy(
                i1_t.at[iix_v.at[pl.ds(lo, _CH)]],
                i1_v.at[pl.ds(lo, _CH)], fsem))
        lane = lax.iota(jnp.int32, 16)
        pos0 = lane * _HID

        def off_body(g, _):
            iu = uix_v[pl.ds(g * 16, 16)]
            ii = iix_v[pl.ds(g * 16, 16)]
            posb = pos0 + g * 16 * _HID
            for c in range(_HID):
                plsc.store_scatter(uoff, [posb + c], iu + c * V)
                plsc.store_scatter(ioff, [posb + c], ii + c * V)
            return ()

        lax.fori_loop(0, b_per_w // 16, off_body, ())

        nch = nel // _CH
        grp = 8

        def fire(j):
            for k in range(grp):
                lo = (j * grp + k) * _CH
                pltpu.async_copy(u2_t.at[uoff.at[pl.ds(lo, _CH)]],
                                 ubuf.at[pl.ds(lo, _CH)], usem)
                pltpu.async_copy(i2_t.at[ioff.at[pl.ds(lo, _CH)]],
                                 ibuf.at[pl.ds(lo, _CH)], isem)

        def drain(j):
            for k in range(grp):
                lo = (j * grp + k) * _CH
                pltpu.make_async_copy(u2_t.at[uoff.at[pl.ds(0, _CH)]],
                                      ubuf.at[pl.ds(lo, _CH)], usem).wait()
                pltpu.make_async_copy(i2_t.at[ioff.at[pl.ds(0, _CH)]],
                                      ibuf.at[pl.ds(lo, _CH)], isem).wait()

        nloop = nch // grp

        def g_body(j, _):
            fire(j)

            @pl.when(j > 0)
            def _():
                drain(j - 1)

            return ()

        lax.fori_loop(0, nloop, g_body, ())
        drain(nloop - 1)
        for c in fcopies:
            c.wait()
        pltpu.sync_copy(ubuf, u2_o.at[pl.ds(base * _HID, nel)])
        pltpu.sync_copy(ibuf, i2_o.at[pl.ds(base * _HID, nel)])
        pltpu.sync_copy(u1_v, u1_o.at[pl.ds(base, b_per_w)])
        pltpu.sync_copy(i1_v, i1_o.at[pl.ds(base, b_per_w)])

    return gath(u2_flat, i2_flat, u1_tab, i1_tab, uidx, iidx)


def _tc_body(xf_ref, u1_ref, i1_ref, u2_ref, i2_ref, bias_ref, r_ref,
             w1_ref, b1_ref, w2_ref, b2_ref, g_ref, gt_ref, c_ref, o_ref):
    blk = xf_ref.shape[0]
    nl = _NF * _FD
    xall = jnp.dot(xf_ref[...].astype(jnp.float32), r_ref[...],
                   preferred_element_type=jnp.float32,
                   precision=lax.Precision.HIGHEST)
    xi = xall.astype(jnp.int32)
    lane = lax.broadcasted_iota(jnp.int32, (blk, nl), 1)
    msk = jnp.left_shift(jnp.int32(1), (_FD - 1) - (lane & (_FD - 1)))
    bits = (jnp.bitwise_and(xi, msk) != 0).astype(jnp.float32)
    bits16 = bits.astype(jnp.bfloat16)
    r2 = jnp.dot(bits16, w2_ref[...], preferred_element_type=jnp.float32)
    r2 = r2 + b2_ref[...]
    sq = r2 * r2
    n2 = jnp.dot(sq, g_ref[...], preferred_element_type=jnp.float32,
                 precision=lax.Precision.HIGHEST)
    inv = 1.0 / jnp.maximum(jnp.sqrt(n2), _EPS)
    invb = jnp.dot(inv, gt_ref[...], preferred_element_type=jnp.float32,
                   precision=lax.Precision.HIGHEST)
    fhat = r2 * invb
    s1 = jnp.dot(bits16, w1_ref[...], preferred_element_type=jnp.float32)
    s1 = s1 + b1_ref[...]
    s1n = s1 / jnp.maximum(jnp.abs(s1), _EPS)
    first_feat = jnp.sum(s1n, axis=1)
    fsum = jnp.dot(fhat, c_ref[...], preferred_element_type=jnp.float32,
                   precision=lax.Precision.HIGHEST)
    u2 = u2_ref[...]
    i2 = i2_ref[...]
    s = u2 + i2 + fsum
    sq_of_sum = jnp.sum(s * s, axis=1)
    sum_of_sq = (jnp.sum(u2 * u2, axis=1) + jnp.sum(i2 * i2, axis=1)
                 + jnp.sum(fhat * fhat, axis=1))
    second = 0.5 * (sq_of_sum - sum_of_sq)
    o_ref[...] = (bias_ref[0] + u1_ref[...] + i1_ref[...] + first_feat
                  + second)


def _tc_combine(xf, u1, i1, u2, i2, bias, rmat, w1, b1, w2, b2, g, gt, c):
    B = xf.shape[0]
    blk = 2048
    grid = (B // blk,)
    full = lambda shape: pl.BlockSpec(shape, lambda i: (0,) * len(shape))
    return pl.pallas_call(
        _tc_body,
        grid=grid,
        in_specs=[
            pl.BlockSpec((blk, _NF), lambda i: (i, 0)),
            pl.BlockSpec((blk,), lambda i: (i,)),
            pl.BlockSpec((blk,), lambda i: (i,)),
            pl.BlockSpec((blk, _HID), lambda i: (i, 0)),
            pl.BlockSpec((blk, _HID), lambda i: (i, 0)),
            pl.BlockSpec(memory_space=pltpu.SMEM),
            full(rmat.shape),
            full(w1.shape),
            full(b1.shape),
            full(w2.shape),
            full(b2.shape),
            full(g.shape),
            full(gt.shape),
            full(c.shape),
        ],
        out_specs=pl.BlockSpec((blk,), lambda i: (i,)),
        out_shape=jax.ShapeDtypeStruct((B,), jnp.float32),
    )(xf, u1, i1, u2, i2, bias, rmat, w1, b1, w2, b2, g, gt, c)


def kernel(x, bias, first_user_emb, first_item_emb, first_feat_w,
           first_feat_b, second_user_emb, second_item_emb, second_feat_w,
           second_feat_b):
    B = x.shape[0]
    V = second_user_emb.shape[0]
    nl = _NF * _FD
    nh = _NF * _HID
    u2f, i2f, u1, i1 = _sc_gather(
        second_user_emb.T.reshape(-1), second_item_emb.T.reshape(-1),
        first_user_emb.reshape(-1), first_item_emb.reshape(-1),
        x[:, 0], x[:, 1], B, V)
    u2 = u2f.reshape(B, _HID)
    i2 = i2f.reshape(B, _HID)
    lane64 = jnp.arange(nl)
    lane128 = jnp.arange(nh)
    rmat = (lane64[None, :] // _FD == jnp.arange(_NF)[:, None]).astype(
        jnp.float32)
    g = (lane128[:, None] // _HID == jnp.arange(_NF)[None, :]).astype(
        jnp.float32)
    gt = jnp.transpose(g)
    c = (lane128[:, None] % _HID == jnp.arange(_HID)[None, :]).astype(
        jnp.float32)
    w2 = (second_feat_w.transpose(0, 2, 1)[:, None, :, :]
          * jnp.eye(_NF)[:, :, None, None])
    w2 = w2.transpose(0, 2, 1, 3).reshape(nl, nh).astype(jnp.bfloat16)
    b2 = second_feat_b.reshape(1, nh)
    w1 = (first_feat_w.transpose(0, 2, 1)[:, None, :, :]
          * jnp.eye(_NF)[:, :, None, None])
    w1 = w1.transpose(0, 2, 1, 3).reshape(nl, _NF).astype(jnp.bfloat16)
    b1 = first_feat_b.reshape(1, _NF)
    xf = x[:, 2:]
    return _tc_combine(xf, u1, i1, u2, i2, bias, rmat, w1, b1, w2, b2, g,
                       gt, c)


# speedup vs baseline: 5.3900x; 5.3900x over previous
import functools

import jax
import jax.numpy as jnp
from jax import lax
from jax.experimental import pallas as pl
from jax.experimental.pallas import tpu as pltpu
from jax.experimental.pallas import tpu_sc as plsc

_NF = 4
_FD = 16
_HID = 32
_EPS = 1e-12
_IDX_CHUNK = 128


def _sc_gather(u2_tab, i2_tab, u1_tab, i1_tab, uidx2d, iidx2d, B):
    info = plsc.get_sparse_core_info()
    nw = info.num_cores * info.num_subcores
    b_per_w = B // nw
    n_chunks = b_per_w // _IDX_CHUNK
    rows_per_w = n_chunks
    mesh = plsc.VectorSubcoreMesh(core_axis_name="c", subcore_axis_name="s")

    @functools.partial(
        pl.kernel,
        mesh=mesh,
        compiler_params=pltpu.CompilerParams(use_tc_tiling_on_sc=False),
        out_type=[
            jax.ShapeDtypeStruct((B, _HID), jnp.float32),
            jax.ShapeDtypeStruct((B, _HID), jnp.float32),
            jax.ShapeDtypeStruct((B,), jnp.float32),
            jax.ShapeDtypeStruct((B,), jnp.float32),
        ],
        scratch_types=[
            pltpu.VMEM((rows_per_w, _IDX_CHUNK), jnp.int32),
            pltpu.VMEM((rows_per_w, _IDX_CHUNK), jnp.int32),
            pltpu.VMEM((b_per_w, _HID), jnp.float32),
            pltpu.VMEM((b_per_w, _HID), jnp.float32),
            pltpu.VMEM((b_per_w,), jnp.float32),
            pltpu.VMEM((b_per_w,), jnp.float32),
            pltpu.SemaphoreType.DMA,
        ],
    )
    def gath(u2_t, i2_t, u1_t, i1_t, uix_h, iix_h, u2_o, i2_o, u1_o, i1_o,
             uix_v, iix_v, u2_v, i2_v, u1_v, i1_v, sem):
        wid = lax.axis_index("s") * info.num_cores + lax.axis_index("c")
        base = wid * b_per_w
        row0 = wid * rows_per_w
        pltpu.sync_copy(uix_h.at[pl.ds(row0, rows_per_w)], uix_v)
        pltpu.sync_copy(iix_h.at[pl.ds(row0, rows_per_w)], iix_v)
        copies = []
        for j in range(n_chunks):
            lo = j * _IDX_CHUNK
            copies.append(pltpu.async_copy(
                u2_t.at[uix_v.at[j]], u2_v.at[pl.ds(lo, _IDX_CHUNK)], sem))
            copies.append(pltpu.async_copy(
                i2_t.at[iix_v.at[j]], i2_v.at[pl.ds(lo, _IDX_CHUNK)], sem))
            copies.append(pltpu.async_copy(
                u1_t.at[uix_v.at[j]], u1_v.at[pl.ds(lo, _IDX_CHUNK)], sem))
            copies.append(pltpu.async_copy(
                i1_t.at[iix_v.at[j]], i1_v.at[pl.ds(lo, _IDX_CHUNK)], sem))
        for c in copies:
            c.wait()
        pltpu.sync_copy(u2_v, u2_o.at[pl.ds(base, b_per_w)])
        pltpu.sync_copy(i2_v, i2_o.at[pl.ds(base, b_per_w)])
        pltpu.sync_copy(u1_v, u1_o.at[pl.ds(base, b_per_w)])
        pltpu.sync_copy(i1_v, i1_o.at[pl.ds(base, b_per_w)])

    return gath(u2_tab, i2_tab, u1_tab, i1_tab, uidx2d, iidx2d)


def _tc_body(xf_ref, u1_ref, i1_ref, u2_ref, i2_ref, bias_ref, r_ref,
             w1_ref, b1_ref, w2_ref, b2_ref, g_ref, gt_ref, c_ref, o_ref):
    blk = xf_ref.shape[0]
    nl = _NF * _FD
    xall = jnp.dot(xf_ref[...].astype(jnp.float32), r_ref[...],
                   preferred_element_type=jnp.float32,
                   precision=lax.Precision.HIGHEST)
    xi = xall.astype(jnp.int32)
    lane = lax.broadcasted_iota(jnp.int32, (blk, nl), 1)
    msk = jnp.left_shift(jnp.int32(1), (_FD - 1) - (lane & (_FD - 1)))
    bits = (jnp.bitwise_and(xi, msk) != 0).astype(jnp.float32)
    bits16 = bits.astype(jnp.bfloat16)
    r2 = jnp.dot(bits16, w2_ref[...], preferred_element_type=jnp.float32)
    r2 = r2 + b2_ref[...]
    sq = r2 * r2
    n2 = jnp.dot(sq, g_ref[...], preferred_element_type=jnp.float32,
                 precision=lax.Precision.HIGHEST)
    inv = 1.0 / jnp.maximum(jnp.sqrt(n2), _EPS)
    invb = jnp.dot(inv, gt_ref[...], preferred_element_type=jnp.float32,
                   precision=lax.Precision.HIGHEST)
    fhat = r2 * invb
    s1 = jnp.dot(bits16, w1_ref[...], preferred_element_type=jnp.float32)
    s1 = s1 + b1_ref[...]
    s1n = s1 / jnp.maximum(jnp.abs(s1), _EPS)
    first_feat = jnp.sum(s1n, axis=1)
    fsum = jnp.dot(fhat, c_ref[...], preferred_element_type=jnp.float32,
                   precision=lax.Precision.HIGHEST)
    u2 = u2_ref[...]
    i2 = i2_ref[...]
    s = u2 + i2 + fsum
    sq_of_sum = jnp.sum(s * s, axis=1)
    sum_of_sq = (jnp.sum(u2 * u2, axis=1) + jnp.sum(i2 * i2, axis=1)
                 + jnp.sum(fhat * fhat, axis=1))
    second = 0.5 * (sq_of_sum - sum_of_sq)
    o_ref[...] = (bias_ref[0] + u1_ref[...] + i1_ref[...] + first_feat
                  + second)


def _tc_combine(xf, u1, i1, u2, i2, bias, rmat, w1, b1, w2, b2, g, gt, c):
    B = xf.shape[0]
    blk = 2048
    grid = (B // blk,)
    full = lambda shape: pl.BlockSpec(shape, lambda i: (0,) * len(shape))
    return pl.pallas_call(
        _tc_body,
        grid=grid,
        in_specs=[
            pl.BlockSpec((blk, _NF), lambda i: (i, 0)),
            pl.BlockSpec((blk,), lambda i: (i,)),
            pl.BlockSpec((blk,), lambda i: (i,)),
            pl.BlockSpec((blk, _HID), lambda i: (i, 0)),
            pl.BlockSpec((blk, _HID), lambda i: (i, 0)),
            pl.BlockSpec(memory_space=pltpu.SMEM),
            full(rmat.shape),
            full(w1.shape),
            full(b1.shape),
            full(w2.shape),
            full(b2.shape),
            full(g.shape),
            full(gt.shape),
            full(c.shape),
        ],
        out_specs=pl.BlockSpec((blk,), lambda i: (i,)),
        out_shape=jax.ShapeDtypeStruct((B,), jnp.float32),
    )(xf, u1, i1, u2, i2, bias, rmat, w1, b1, w2, b2, g, gt, c)


def kernel(x, bias, first_user_emb, first_item_emb, first_feat_w,
           first_feat_b, second_user_emb, second_item_emb, second_feat_w,
           second_feat_b):
    B = x.shape[0]
    nl = _NF * _FD
    nh = _NF * _HID
    uidx2d = x[:, 0].reshape(B // _IDX_CHUNK, _IDX_CHUNK)
    iidx2d = x[:, 1].reshape(B // _IDX_CHUNK, _IDX_CHUNK)
    u2, i2, u1, i1 = _sc_gather(
        second_user_emb, second_item_emb,
        first_user_emb.reshape(-1), first_item_emb.reshape(-1),
        uidx2d, iidx2d, B)
    lane64 = jnp.arange(nl)
    lane128 = jnp.arange(nh)
    rmat = (lane64[None, :] // _FD == jnp.arange(_NF)[:, None]).astype(
        jnp.float32)
    g = (lane128[:, None] // _HID == jnp.arange(_NF)[None, :]).astype(
        jnp.float32)
    gt = jnp.transpose(g)
    c = (lane128[:, None] % _HID == jnp.arange(_HID)[None, :]).astype(
        jnp.float32)
    w2 = (second_feat_w.transpose(0, 2, 1)[:, None, :, :]
          * jnp.eye(_NF)[:, :, None, None])
    w2 = w2.transpose(0, 2, 1, 3).reshape(nl, nh).astype(jnp.bfloat16)
    b2 = second_feat_b.reshape(1, nh)
    w1 = (first_feat_w.transpose(0, 2, 1)[:, None, :, :]
          * jnp.eye(_NF)[:, :, None, None])
    w1 = w1.transpose(0, 2, 1, 3).reshape(nl, _NF).astype(jnp.bfloat16)
    b1 = first_feat_b.reshape(1, _NF)
    xf = x[:, 2:]
    return _tc_combine(xf, u1, i1, u2, i2, bias, rmat, w1, b1, w2, b2, g,
                       gt, c)
